# initial kernel scaffold (unmeasured)
import jax
import jax.numpy as jnp
from jax import lax
from jax.experimental import pallas as pl
from jax.experimental.pallas import tpu as pltpu

N_DEV = 8


def _a2a(x_shard):
    m_glob, k_per = x_shard.shape
    m_per = m_glob // N_DEV

    def body(x_ref, out_ref, send_sems, recv_sems):
        my = lax.axis_index("i")

        out_ref[:, pl.ds(my * k_per, k_per)] = x_ref[pl.ds(my * m_per, m_per), :]

        rdmas = []
        for off in range(1, N_DEV):
            dst = lax.rem(my + off, N_DEV)
            rdma = pltpu.make_async_remote_copy(
                src_ref=x_ref.at[pl.ds(dst * m_per, m_per), :],
                dst_ref=out_ref.at[:, pl.ds(my * k_per, k_per)],
                send_sem=send_sems.at[off],
                recv_sem=recv_sems.at[off],
                device_id=(dst,),
                device_id_type=pl.DeviceIdType.MESH,
            )
            rdma.start()
            rdmas.append(rdma)
        for rdma in rdmas:
            rdma.wait()

    return pl.pallas_call(
        body,
        out_shape=jax.ShapeDtypeStruct((m_per, m_glob), x_shard.dtype),
        in_specs=[pl.BlockSpec(memory_space=pltpu.VMEM)],
        out_specs=pl.BlockSpec(memory_space=pltpu.VMEM),
        scratch_shapes=[
            pltpu.SemaphoreType.DMA((N_DEV,)),
            pltpu.SemaphoreType.DMA((N_DEV,)),
        ],
    )(x_shard)


def _gemm(xg, w):
    m, k_glob = xg.shape
    _, n = w.shape
    blk = k_glob // N_DEV

    def body(x_ref, w_ref, o_ref):
        @pl.when(pl.program_id(0) == 0)
        def _():
            o_ref[...] = jnp.zeros_like(o_ref)

        o_ref[...] += jnp.dot(
            x_ref[...], w_ref[...], preferred_element_type=jnp.float32
        )

    return pl.pallas_call(
        body,
        grid=(N_DEV,),
        in_specs=[
            pl.BlockSpec((m, blk), lambda k: (0, k)),
            pl.BlockSpec((blk, n), lambda k: (k, 0)),
        ],
        out_specs=pl.BlockSpec((m, n), lambda k: (0, 0)),
        out_shape=jax.ShapeDtypeStruct((m, n), jnp.float32),
        compiler_params=pltpu.CompilerParams(
            dimension_semantics=("arbitrary",)
        ),
    )(xg, w)


def kernel(x, w_mat):
    x = x.astype(jnp.bfloat16)
    w_mat = w_mat.astype(jnp.bfloat16)
    xg = _a2a(x)
    return _gemm(xg, w_mat)


# baseline (device time: 321533 ns/iter reference)
import jax
import jax.numpy as jnp
from jax import lax
from jax.experimental import pallas as pl
from jax.experimental.pallas import tpu as pltpu

N_DEV = 8


def _a2a(x_shard):
    m_glob, k_per = x_shard.shape
    m_per = m_glob // N_DEV

    def body(x_ref, out_ref, send_sems, recv_sems):
        my = lax.axis_index("i")

        out_ref[:, pl.ds(my * k_per, k_per)] = x_ref[pl.ds(my * m_per, m_per), :]

        rdmas = []
        for off in range(1, N_DEV):
            dst = lax.rem(my + off, N_DEV)
            rdma = pltpu.make_async_remote_copy(
                src_ref=x_ref.at[pl.ds(dst * m_per, m_per), :],
                dst_ref=out_ref.at[:, pl.ds(my * k_per, k_per)],
                send_sem=send_sems.at[off],
                recv_sem=recv_sems.at[off],
                device_id=(dst,),
                device_id_type=pl.DeviceIdType.MESH,
            )
            rdma.start()
            rdmas.append(rdma)
        for rdma in rdmas:
            rdma.wait()

    return pl.pallas_call(
        body,
        out_shape=jax.ShapeDtypeStruct((m_per, m_glob), x_shard.dtype),
        in_specs=[pl.BlockSpec(memory_space=pltpu.VMEM)],
        out_specs=pl.BlockSpec(memory_space=pltpu.VMEM),
        scratch_shapes=[
            pltpu.SemaphoreType.DMA((N_DEV,)),
            pltpu.SemaphoreType.DMA((N_DEV,)),
        ],
        compiler_params=pltpu.CompilerParams(
            vmem_limit_bytes=100 * 1024 * 1024
        ),
    )(x_shard)


def _gemm(xg, w):
    m, k_glob = xg.shape
    _, n = w.shape
    blk = k_glob // N_DEV

    def body(x_ref, w_ref, o_ref):
        @pl.when(pl.program_id(0) == 0)
        def _():
            o_ref[...] = jnp.zeros_like(o_ref)

        o_ref[...] += jnp.dot(
            x_ref[...], w_ref[...], preferred_element_type=jnp.float32
        )

    return pl.pallas_call(
        body,
        grid=(N_DEV,),
        in_specs=[
            pl.BlockSpec((m, blk), lambda k: (0, k)),
            pl.BlockSpec((blk, n), lambda k: (k, 0)),
        ],
        out_specs=pl.BlockSpec((m, n), lambda k: (0, 0)),
        out_shape=jax.ShapeDtypeStruct((m, n), jnp.float32),
        compiler_params=pltpu.CompilerParams(
            dimension_semantics=("arbitrary",),
            vmem_limit_bytes=100 * 1024 * 1024,
        ),
    )(xg, w)


def kernel(x, w_mat):
    x = x.astype(jnp.bfloat16)
    w_mat = w_mat.astype(jnp.bfloat16)
    xg = _a2a(x)
    return _gemm(xg, w_mat)


# device time: 200485 ns/iter; 1.6038x vs baseline; 1.6038x over previous
import jax
import jax.numpy as jnp
from jax import lax
from jax.experimental import pallas as pl
from jax.experimental.pallas import tpu as pltpu

N_DEV = 8
M_PER = 1024
K_PER = 1024
N_OUT = 4096
W_CHUNK = 256
CPO = K_PER // W_CHUNK
N_CHUNKS = CPO * N_DEV


def _fused(x_shard, w_mat):

    def body(x_ref, w_ref, o_ref, xg_ref, wf_ref, wb_ref,
             wdma_sems, send_sems, recv_sems):
        my = lax.axis_index("i")

        barrier_sem = pltpu.get_barrier_semaphore()
        for off in range(1, N_DEV):
            pl.semaphore_signal(
                barrier_sem, inc=1,
                device_id=(lax.rem(my + off, N_DEV),),
                device_id_type=pl.DeviceIdType.MESH,
            )
        pl.semaphore_wait(barrier_sem, N_DEV - 1)

        def w_chunk_copy(slot, t, c):
            src = lax.rem(my + (N_DEV - t), N_DEV)
            row0 = src * K_PER + c * W_CHUNK
            return pltpu.make_async_copy(
                w_ref.at[pl.ds(row0, W_CHUNK), :],
                wf_ref.at[slot],
                wdma_sems.at[slot],
            )

        w_chunk_copy(0, 0, 0).start()
        w_chunk_copy(1, 0, 1).start()

        rdmas = []
        for off in range(1, N_DEV):
            dst = lax.rem(my + off, N_DEV)
            rdma = pltpu.make_async_remote_copy(
                src_ref=x_ref.at[pl.ds(dst * M_PER, M_PER), :],
                dst_ref=xg_ref.at[off],
                send_sem=send_sems.at[off],
                recv_sem=recv_sems.at[off],
                device_id=(dst,),
                device_id_type=pl.DeviceIdType.MESH,
            )
            rdma.start()
            rdmas.append(rdma)

        xg_ref[0] = x_ref[pl.ds(my * M_PER, M_PER), :]

        def origin_step(t, is_first):
            if not is_first:
                pltpu.make_async_remote_copy(
                    src_ref=x_ref.at[pl.ds(0, M_PER), :],
                    dst_ref=xg_ref.at[t],
                    send_sem=send_sems.at[0],
                    recv_sem=recv_sems.at[t],
                    device_id=(my,),
                    device_id_type=pl.DeviceIdType.MESH,
                ).wait_recv()
            for c in range(CPO):
                slot = c % 2
                w_chunk_copy(slot, t, c).wait()
                wb_ref[slot] = wf_ref[slot].astype(jnp.bfloat16)
                nt, nc = (t, c + 2) if c + 2 < CPO else (t + 1, c + 2 - CPO)
                if c + 2 < CPO:
                    w_chunk_copy(slot, nt, nc).start()
                else:
                    @pl.when(t != N_DEV - 1)
                    def _():
                        w_chunk_copy(slot, nt, nc).start()
                xb = xg_ref[t, :, c * W_CHUNK:(c + 1) * W_CHUNK]
                acc = jnp.dot(xb, wb_ref[slot],
                              preferred_element_type=jnp.float32)
                if is_first and c == 0:
                    o_ref[...] = acc
                else:
                    o_ref[...] += acc

        origin_step(0, True)

        def loop_body(t, _):
            origin_step(t, False)
            return ()

        lax.fori_loop(1, N_DEV, loop_body, (), unroll=False)

        for rdma in rdmas:
            rdma.wait_send()

    return pl.pallas_call(
        body,
        out_shape=jax.ShapeDtypeStruct((M_PER, N_OUT), jnp.float32),
        in_specs=[
            pl.BlockSpec(memory_space=pltpu.VMEM),
            pl.BlockSpec(memory_space=pltpu.MemorySpace.HBM),
        ],
        out_specs=pl.BlockSpec(memory_space=pltpu.VMEM),
        scratch_shapes=[
            pltpu.VMEM((N_DEV, M_PER, K_PER), jnp.bfloat16),
            pltpu.VMEM((2, W_CHUNK, N_OUT), jnp.float32),
            pltpu.VMEM((2, W_CHUNK, N_OUT), jnp.bfloat16),
            pltpu.SemaphoreType.DMA((2,)),
            pltpu.SemaphoreType.DMA((N_DEV,)),
            pltpu.SemaphoreType.DMA((N_DEV,)),
        ],
        compiler_params=pltpu.CompilerParams(
            collective_id=0,
            vmem_limit_bytes=110 * 1024 * 1024,
        ),
    )(x_shard, w_mat)


def kernel(x, w_mat):
    x = x.astype(jnp.bfloat16)
    return _fused(x, w_mat)
